# baseline (speedup 1.0000x reference)
import functools
import math

import jax
import jax.numpy as jnp
from jax import lax
from jax.experimental import pallas as pl
from jax.experimental.pallas import tpu as pltpu
from jax.experimental.pallas import tpu_sc as plsc

D_MODEL = 32
SCALE = math.sqrt(D_MODEL)

NC = 2
NS = 16
NW = NC * NS

CHUNK = 128
NBUF = 8

TBLK = 8192


def _tc_relayout(lut_t):
    vocab = lut_t.shape[1]
    grid = (vocab + TBLK - 1) // TBLK

    def body(l_ref, o_ref):
        o_ref[...] = l_ref[...].T * SCALE

    return pl.pallas_call(
        body,
        grid=(grid,),
        in_specs=[pl.BlockSpec((D_MODEL, TBLK), lambda i: (0, i))],
        out_specs=pl.BlockSpec((TBLK, D_MODEL), lambda i: (i, 0)),
        out_shape=jax.ShapeDtypeStruct((vocab, D_MODEL), jnp.float32),
    )(lut_t)


def _make_sc_gather(n_j):
    out_shape = (NW * n_j * CHUNK, D_MODEL)

    @functools.partial(
        pl.kernel,
        out_type=jax.ShapeDtypeStruct(out_shape, jnp.float32),
        mesh=plsc.VectorSubcoreMesh(core_axis_name="c", subcore_axis_name="s"),
        scratch_types=[
            pltpu.VMEM((n_j, CHUNK), jnp.int32),
            pltpu.VMEM((NBUF, CHUNK, D_MODEL), jnp.float32),
            pltpu.VMEM((NBUF, 4, 8, CHUNK), jnp.float32),
        ]
        + [pltpu.SemaphoreType.DMA] * (2 * NBUF),
        compiler_params=pltpu.CompilerParams(
            use_tc_tiling_on_sc=False, needs_layout_passes=False
        ),
    )
    def body(idx_hbm, table_hbm, flat_hbm, idx_v, gbuf, obuf, *sems):
        gsems = sems[:NBUF]
        osems = sems[NBUF:]
        c = lax.axis_index("c")
        s = lax.axis_index("s")
        wid = s * NC + c
        pltpu.sync_copy(idx_hbm.at[wid], idx_v)

        def issue_gather(g, b):
            pltpu.async_copy(table_hbm.at[idx_v.at[g]], gbuf.at[b], gsems[b])

        def wait_gather(g, b):
            pltpu.make_async_copy(
                table_hbm.at[idx_v.at[g]], gbuf.at[b], gsems[b]
            ).wait()

        def issue_out(g, b):
            pltpu.async_copy(
                gbuf.at[b],
                flat_hbm.at[pl.ds((wid * n_j + g) * CHUNK, CHUNK)],
                osems[b],
            )

        def wait_out(g, b):
            pltpu.make_async_copy(
                gbuf.at[b],
                flat_hbm.at[pl.ds((wid * n_j + g) * CHUNK, CHUNK)],
                osems[b],
            ).wait()

        iota = lax.iota(jnp.int32, 16)
        rows_c = [iota + 16 * icg for icg in range(CHUNK // 16)]
        cols_c = [jnp.full((16,), k, jnp.int32) for k in range(D_MODEL)]

        def transpose_chunk(b):
            for k in range(D_MODEL):
                for icg in range(CHUNK // 16):
                    vals = plsc.load_gather(
                        gbuf.at[b], [rows_c[icg], cols_c[k]]
                    )
                    obuf[b, k // 8, k % 8, pl.ds(icg * 16, 16)] = vals

        for b in range(NBUF):
            issue_gather(b, b)

        niter = n_j // NBUF

        def mid(i, carry):
            for b in range(NBUF):
                g = i * NBUF + b
                wait_gather(g, b)

                @pl.when(g >= NBUF)
                def _():
                    wait_out(g - NBUF, b)


                @pl.when(g + NBUF < n_j)
                def _():
                    issue_gather(g + NBUF, b)

                issue_out(g, b)
            return carry

        lax.fori_loop(0, niter, mid, 0)

        for b in range(NBUF):
            wait_out((niter - 1) * NBUF + b, b)

    return body


def kernel(x, lut):
    n_i, n_j = x.shape
    table = _tc_relayout(lut.T)
    xi = jnp.transpose(
        jnp.asarray(x, jnp.int32).T.reshape(n_j, NW, CHUNK), (1, 0, 2)
    )
    outf = _make_sc_gather(n_j)(xi, table)
    return outf.reshape(NW, n_j, CHUNK, D_MODEL).transpose(2, 1, 0, 3).reshape(
        n_i, n_j, D_MODEL)


# speedup vs baseline: 1.2298x; 1.2298x over previous
import functools
import math

import jax
import jax.numpy as jnp
from jax import lax
from jax.experimental import pallas as pl
from jax.experimental.pallas import tpu as pltpu
from jax.experimental.pallas import tpu_sc as plsc

D_MODEL = 32
SCALE = math.sqrt(D_MODEL)

NC = 2
NS = 16
NW = NC * NS

CHUNK = 128
NBUF = 8

TBLK = 8192


def _tc_relayout(lut_t):
    vocab = lut_t.shape[1]
    grid = (vocab + TBLK - 1) // TBLK

    def body(l_ref, o_ref):
        o_ref[...] = l_ref[...].T * SCALE

    return pl.pallas_call(
        body,
        grid=(grid,),
        in_specs=[pl.BlockSpec((D_MODEL, TBLK), lambda i: (0, i))],
        out_specs=pl.BlockSpec((TBLK, D_MODEL), lambda i: (i, 0)),
        out_shape=jax.ShapeDtypeStruct((vocab, D_MODEL), jnp.float32),
    )(lut_t)


def _make_sc_gather(n_j):
    out_shape = (n_j, 4, NW, 8, CHUNK)

    @functools.partial(
        pl.kernel,
        out_type=jax.ShapeDtypeStruct(out_shape, jnp.float32),
        mesh=plsc.VectorSubcoreMesh(core_axis_name="c", subcore_axis_name="s"),
        scratch_types=[
            pltpu.VMEM((n_j, CHUNK), jnp.int32),
            pltpu.VMEM((NBUF, CHUNK, D_MODEL), jnp.float32),
            pltpu.VMEM((NBUF, 4, 8, CHUNK), jnp.float32),
        ]
        + [pltpu.SemaphoreType.DMA] * (2 * NBUF),
        compiler_params=pltpu.CompilerParams(
            use_tc_tiling_on_sc=False, needs_layout_passes=False
        ),
    )
    def body(idx_hbm, table_hbm, out_hbm, idx_v, gbuf, obuf, *sems):
        gsems = sems[:NBUF]
        osems = sems[NBUF:]
        c = lax.axis_index("c")
        s = lax.axis_index("s")
        wid = s * NC + c
        pltpu.sync_copy(idx_hbm.at[wid], idx_v)

        def issue_gather(g, b):
            pltpu.async_copy(table_hbm.at[idx_v.at[g]], gbuf.at[b], gsems[b])

        def wait_gather(g, b):
            pltpu.make_async_copy(
                table_hbm.at[idx_v.at[g]], gbuf.at[b], gsems[b]
            ).wait()

        def issue_out(g, b):
            for cb in range(4):
                pltpu.async_copy(
                    obuf.at[b, cb], out_hbm.at[g, cb, wid], osems[b]
                )

        def wait_out(g, b):
            for cb in range(4):
                pltpu.make_async_copy(
                    obuf.at[b, cb], out_hbm.at[g, cb, wid], osems[b]
                ).wait()

        iota = lax.iota(jnp.int32, 16)
        rows_c = [iota + 16 * icg for icg in range(CHUNK // 16)]
        cols_c = [jnp.full((16,), k, jnp.int32) for k in range(D_MODEL)]

        def transpose_chunk(b):
            for k in range(D_MODEL):
                for icg in range(CHUNK // 16):
                    vals = plsc.load_gather(
                        gbuf.at[b], [rows_c[icg], cols_c[k]]
                    )
                    obuf[b, k // 8, k % 8, pl.ds(icg * 16, 16)] = vals

        for b in range(NBUF):
            issue_gather(b, b)

        niter = n_j // NBUF

        def mid(i, carry):
            for b in range(NBUF):
                g = i * NBUF + b
                wait_gather(g, b)

                @pl.when(g >= NBUF)
                def _():
                    wait_out(g - NBUF, b)

                transpose_chunk(b)

                @pl.when(g + NBUF < n_j)
                def _():
                    issue_gather(g + NBUF, b)

                issue_out(g, b)
            return carry

        lax.fori_loop(0, niter, mid, 0)

        for b in range(NBUF):
            wait_out((niter - 1) * NBUF + b, b)

    return body


def kernel(x, lut):
    n_i, n_j = x.shape
    table = _tc_relayout(lut.T)
    xi = jnp.transpose(
        jnp.asarray(x, jnp.int32).T.reshape(n_j, NW, CHUNK), (1, 0, 2)
    )
    out5 = _make_sc_gather(n_j)(xi, table)
    return out5.transpose(2, 4, 0, 1, 3).reshape(n_i, n_j, D_MODEL)


# speedup vs baseline: 1.5055x; 1.2241x over previous
import functools
import math

import jax
import jax.numpy as jnp
from jax import lax
from jax.experimental import pallas as pl
from jax.experimental.pallas import tpu as pltpu
from jax.experimental.pallas import tpu_sc as plsc

D_MODEL = 32
SCALE = math.sqrt(D_MODEL)

NC = 2
NS = 16
NW = NC * NS

CHUNK = 128
NBUF = 8

TBLK = 8192


def _tc_relayout(lut_t):
    vocab = lut_t.shape[1]
    grid = (vocab + TBLK - 1) // TBLK

    def body(l_ref, o_ref):
        o_ref[...] = l_ref[...].T * SCALE

    return pl.pallas_call(
        body,
        grid=(grid,),
        in_specs=[pl.BlockSpec((D_MODEL, TBLK), lambda i: (0, i))],
        out_specs=pl.BlockSpec((TBLK, D_MODEL), lambda i: (i, 0)),
        out_shape=jax.ShapeDtypeStruct((vocab, D_MODEL), jnp.float32),
    )(lut_t)


def _make_sc_gather(n_j):
    out_shape = (n_j, 4, NW, 8 * CHUNK)

    @functools.partial(
        pl.kernel,
        out_type=jax.ShapeDtypeStruct(out_shape, jnp.float32),
        mesh=plsc.VectorSubcoreMesh(core_axis_name="c", subcore_axis_name="s"),
        scratch_types=[
            pltpu.VMEM((n_j, CHUNK), jnp.int32),
            pltpu.VMEM((NBUF, CHUNK, D_MODEL), jnp.float32),
            pltpu.VMEM((NBUF, D_MODEL * CHUNK), jnp.float32),
        ]
        + [pltpu.SemaphoreType.DMA] * (2 * NBUF),
        compiler_params=pltpu.CompilerParams(
            use_tc_tiling_on_sc=False, needs_layout_passes=False
        ),
    )
    def body(idx_hbm, table_hbm, out_hbm, idx_v, gbuf, obuf, *sems):
        gsems = sems[:NBUF]
        osems = sems[NBUF:]
        c = lax.axis_index("c")
        s = lax.axis_index("s")
        wid = s * NC + c
        pltpu.sync_copy(idx_hbm.at[wid], idx_v)

        def issue_gather(g, b):
            pltpu.async_copy(table_hbm.at[idx_v.at[g]], gbuf.at[b], gsems[b])

        def wait_gather(g, b):
            pltpu.make_async_copy(
                table_hbm.at[idx_v.at[g]], gbuf.at[b], gsems[b]
            ).wait()

        def issue_out(g, b):
            for cb in range(4):
                pltpu.async_copy(
                    obuf.at[b, pl.ds(cb * 8 * CHUNK, 8 * CHUNK)],
                    out_hbm.at[g, cb, wid],
                    osems[b],
                )

        def wait_out(g, b):
            for cb in range(4):
                pltpu.make_async_copy(
                    obuf.at[b, pl.ds(cb * 8 * CHUNK, 8 * CHUNK)],
                    out_hbm.at[g, cb, wid],
                    osems[b],
                ).wait()

        scat_base = lax.iota(jnp.int32, 16) * CHUNK

        def transpose_chunk(b):
            def row_body(ic, carry):
                for h in range(2):
                    vals = gbuf[b, ic, pl.ds(h * 16, 16)]
                    idxv = scat_base + (h * 16 * CHUNK + ic)
                    plsc.store_scatter(obuf.at[b], [idxv], vals)
                return carry

            lax.fori_loop(0, CHUNK, row_body, 0, unroll=8)

        for b in range(NBUF):
            issue_gather(b, b)

        niter = n_j // NBUF

        def mid(i, carry):
            for b in range(NBUF):
                g = i * NBUF + b
                wait_gather(g, b)

                @pl.when(g >= NBUF)
                def _():
                    wait_out(g - NBUF, b)

                transpose_chunk(b)

                @pl.when(g + NBUF < n_j)
                def _():
                    issue_gather(g + NBUF, b)

                issue_out(g, b)
            return carry

        lax.fori_loop(0, niter, mid, 0)

        for b in range(NBUF):
            wait_out((niter - 1) * NBUF + b, b)

    return body


def kernel(x, lut):
    n_i, n_j = x.shape
    table = _tc_relayout(lut.T)
    xi = jnp.transpose(
        jnp.asarray(x, jnp.int32).T.reshape(n_j, NW, CHUNK), (1, 0, 2)
    )
    out4 = _make_sc_gather(n_j)(xi, table)
    out5 = out4.reshape(n_j, 4, NW, 8, CHUNK)
    return out5.transpose(2, 4, 0, 1, 3).reshape(n_i, n_j, D_MODEL)
